# baseline (speedup 1.0000x reference)
import functools

import jax
import jax.numpy as jnp
from jax import lax
from jax.experimental import pallas as pl
from jax.experimental.pallas import tpu as pltpu
from jax.experimental.pallas import tpu_sc as plsc

N_NODES = 10000
D = 128
N_EDGES = 320000

NC = 2
NS = 16
NW = NC * NS
PW = D // 2

E_PER_T = N_EDGES // NW
B = 400
NBK = E_PER_T // B


def _mlp_body(x_ref, w1_ref, b1_ref, w2_ref, b2_ref, w_ref):
    h1 = jnp.dot(x_ref[...], w1_ref[...], preferred_element_type=jnp.float32)
    h1 = jnp.maximum(h1 + b1_ref[...], 0.0)
    h2 = jnp.dot(h1, w2_ref[...], preferred_element_type=jnp.float32)
    h2 = h2 + b2_ref[...]
    hb = h2.astype(jnp.bfloat16)
    lo = lax.bitcast_convert_type(hb[:, :PW], jnp.uint16).astype(jnp.uint32)
    hi = lax.bitcast_convert_type(hb[:, PW:], jnp.uint16).astype(jnp.uint32)
    w_ref[...] = lax.bitcast_convert_type(lo | (hi << 16), jnp.float32)


def _mlp_packed(node_feature, W1, b1, W2, b2):
    return pl.pallas_call(
        _mlp_body,
        out_shape=jax.ShapeDtypeStruct((N_NODES, PW), jnp.float32),
    )(node_feature, W1, b1.reshape(1, D), W2, b2.reshape(1, D))


_sc_mesh = plsc.VectorSubcoreMesh(core_axis_name="c", subcore_axis_name="s")


@functools.partial(
    pl.kernel,
    out_type=jax.ShapeDtypeStruct((N_EDGES,), jnp.float32),
    mesh=_sc_mesh,
    scratch_types=[
        pltpu.VMEM((2 * B, PW), jnp.float32),
        pltpu.VMEM((2 * B, PW), jnp.float32),
        pltpu.VMEM((2 * B,), jnp.int32),
        pltpu.VMEM((2 * B,), jnp.int32),
        pltpu.VMEM((2 * B,), jnp.float32),
        pltpu.SemaphoreType.DMA,
        pltpu.SemaphoreType.DMA,
        pltpu.SemaphoreType.DMA,
        pltpu.SemaphoreType.DMA,
        pltpu.SemaphoreType.DMA,
    ],
    compiler_params=pltpu.CompilerParams(needs_layout_passes=False,
                                         use_tc_tiling_on_sc=False),
)
def _sc_edge_dot(h_packed, idx_flat, out_hbm,
                 rows0_v, rows1_v, idx0_v, idx1_v, out_v,
                 sem_i0, sem_i1, sem_r0, sem_r1, sem_o):
    t = lax.axis_index("s") * NC + lax.axis_index("c")
    iota16 = lax.iota(jnp.int32, 16)
    last_lane = iota16 == 15
    tbase = t * E_PER_T

    def idx_copies(kb, buf):
        src0 = idx_flat.at[pl.ds(tbase + kb * B, B)]
        src1 = idx_flat.at[pl.ds(N_EDGES + tbase + kb * B, B)]
        d0 = idx0_v.at[pl.ds(buf * B, B)]
        d1 = idx1_v.at[pl.ds(buf * B, B)]
        return ((src0, d0, sem_i0), (src1, d1, sem_i1))

    def row_copies(buf):
        i0 = idx0_v.at[pl.ds(buf * B, B)]
        i1 = idx1_v.at[pl.ds(buf * B, B)]
        return ((h_packed.at[i0], rows0_v.at[pl.ds(buf * B, B)], sem_r0),
                (h_packed.at[i1], rows1_v.at[pl.ds(buf * B, B)], sem_r1))

    for s, d, sem in idx_copies(0, 0):
        pltpu.async_copy(s, d, sem)
    for s, d, sem in idx_copies(0, 0):
        pltpu.make_async_copy(s, d, sem).wait()
    for s, d, sem in row_copies(0):
        pltpu.async_copy(s, d, sem)
    for s, d, sem in idx_copies(1, 1):
        pltpu.async_copy(s, d, sem)

    def block_body(kb, _):
        b = lax.rem(kb, 2)
        nb = 1 - b

        @pl.when(kb + 1 < NBK)
        def _start_next_rows():
            for s, d, sem in idx_copies(kb + 1, nb):
                pltpu.make_async_copy(s, d, sem).wait()
            for s, d, sem in row_copies(nb):
                pltpu.async_copy(s, d, sem)

        for s, d, sem in row_copies(b):
            pltpu.make_async_copy(s, d, sem).wait()

        @pl.when(kb + 2 < NBK)
        def _prefetch_idx():
            for s, d, sem in idx_copies(kb + 2, b):
                pltpu.async_copy(s, d, sem)

        boff = b * B

        @plsc.parallel_loop(0, B, 1, unroll=8)
        def ebody(e):
            acc = jnp.zeros((16,), jnp.float32)
            for c in range(PW // 16):
                wa = rows0_v[boff + e, pl.ds(c * 16, 16)]
                wb = rows1_v[boff + e, pl.ds(c * 16, 16)]
                m = plsc.bitcast(wa, jnp.bfloat16) * plsc.bitcast(
                    wb, jnp.bfloat16)
                m0, m1 = plsc.unpack(m, format=plsc.PackFormat.INTERLEAVED)
                acc = acc + m0 + m1
            tot = plsc.cumsum(acc)
            plsc.store_scatter(out_v, [jnp.full((16,), boff + e, jnp.int32)],
                               tot, mask=last_lane)

        @pl.when(kb >= 1)
        def _wait_prev_out():
            pltpu.make_async_copy(out_v.at[pl.ds(nb * B, B)],
                                  out_hbm.at[pl.ds(tbase, B)], sem_o).wait()
        pltpu.async_copy(out_v.at[pl.ds(boff, B)],
                         out_hbm.at[pl.ds(tbase + kb * B, B)], sem_o)
        return 0

    lax.fori_loop(0, NBK, block_body, 0)
    pltpu.make_async_copy(out_v.at[pl.ds(lax.rem(NBK - 1, 2) * B, B)],
                          out_hbm.at[pl.ds(tbase, B)], sem_o).wait()


def kernel(node_feature, edge_label_index, edge_label, W1, b1, W2, b2):
    h_packed = _mlp_packed(node_feature, W1, b1, W2, b2)
    pred = _sc_edge_dot(h_packed, edge_label_index.reshape(-1))
    return pred, edge_label


# speedup vs baseline: 1.0062x; 1.0062x over previous
import functools

import jax
import jax.numpy as jnp
from jax import lax
from jax.experimental import pallas as pl
from jax.experimental.pallas import tpu as pltpu
from jax.experimental.pallas import tpu_sc as plsc

N_NODES = 10000
D = 128
N_EDGES = 320000

NC = 2
NS = 16
NW = NC * NS
PW = D // 2

E_PER_T = N_EDGES // NW
B = 200
NBK = E_PER_T // B
NBUF = 3


def _mlp_body(x_ref, w1_ref, b1_ref, w2_ref, b2_ref, w_ref):
    h1 = jnp.dot(x_ref[...], w1_ref[...], preferred_element_type=jnp.float32)
    h1 = jnp.maximum(h1 + b1_ref[...], 0.0)
    h2 = jnp.dot(h1, w2_ref[...], preferred_element_type=jnp.float32)
    h2 = h2 + b2_ref[...]
    hb = h2.astype(jnp.bfloat16)
    lo = lax.bitcast_convert_type(hb[:, :PW], jnp.uint16).astype(jnp.uint32)
    hi = lax.bitcast_convert_type(hb[:, PW:], jnp.uint16).astype(jnp.uint32)
    w_ref[...] = lax.bitcast_convert_type(lo | (hi << 16), jnp.float32)


def _mlp_packed(node_feature, W1, b1, W2, b2):
    return pl.pallas_call(
        _mlp_body,
        out_shape=jax.ShapeDtypeStruct((N_NODES, PW), jnp.float32),
    )(node_feature, W1, b1.reshape(1, D), W2, b2.reshape(1, D))


_sc_mesh = plsc.VectorSubcoreMesh(core_axis_name="c", subcore_axis_name="s")


@functools.partial(
    pl.kernel,
    out_type=jax.ShapeDtypeStruct((N_EDGES,), jnp.float32),
    mesh=_sc_mesh,
    scratch_types=[
        pltpu.VMEM((NBUF * B, PW), jnp.float32),
        pltpu.VMEM((NBUF * B, PW), jnp.float32),
        pltpu.VMEM((NBUF * B,), jnp.int32),
        pltpu.VMEM((NBUF * B,), jnp.int32),
        pltpu.VMEM((2 * B,), jnp.float32),
        pltpu.SemaphoreType.DMA,
        pltpu.SemaphoreType.DMA,
        pltpu.SemaphoreType.DMA,
        pltpu.SemaphoreType.DMA,
        pltpu.SemaphoreType.DMA,
    ],
    compiler_params=pltpu.CompilerParams(needs_layout_passes=False,
                                         use_tc_tiling_on_sc=False),
)
def _sc_edge_dot(h_packed, idx_flat, out_hbm,
                 rows0_v, rows1_v, idx0_v, idx1_v, out_v,
                 sem_i0, sem_i1, sem_r0, sem_r1, sem_o):
    t = lax.axis_index("s") * NC + lax.axis_index("c")
    iota16 = lax.iota(jnp.int32, 16)
    last_lane = iota16 == 15
    tbase = t * E_PER_T

    def idx_copies(kb, buf):
        src0 = idx_flat.at[pl.ds(tbase + kb * B, B)]
        src1 = idx_flat.at[pl.ds(N_EDGES + tbase + kb * B, B)]
        d0 = idx0_v.at[pl.ds(buf * B, B)]
        d1 = idx1_v.at[pl.ds(buf * B, B)]
        return ((src0, d0, sem_i0), (src1, d1, sem_i1))

    def row_copies(buf):
        i0 = idx0_v.at[pl.ds(buf * B, B)]
        i1 = idx1_v.at[pl.ds(buf * B, B)]
        return ((h_packed.at[i0], rows0_v.at[pl.ds(buf * B, B)], sem_r0),
                (h_packed.at[i1], rows1_v.at[pl.ds(buf * B, B)], sem_r1))

    for k in range(NBUF):
        for s, d, sem in idx_copies(k, k):
            pltpu.async_copy(s, d, sem)
    for k in range(2):
        for s, d, sem in idx_copies(k, k):
            pltpu.make_async_copy(s, d, sem).wait()
        for s, d, sem in row_copies(k):
            pltpu.async_copy(s, d, sem)

    def block_body(kb, _):
        b = lax.rem(kb, NBUF)
        b2 = lax.rem(kb + 2, NBUF)

        @pl.when(kb + 2 < NBK)
        def _start_ahead_rows():
            for s, d, sem in idx_copies(kb + 2, b2):
                pltpu.make_async_copy(s, d, sem).wait()
            for s, d, sem in row_copies(b2):
                pltpu.async_copy(s, d, sem)

        for s, d, sem in row_copies(b):
            pltpu.make_async_copy(s, d, sem).wait()

        @pl.when(kb + NBUF < NBK)
        def _prefetch_idx():
            for s, d, sem in idx_copies(kb + NBUF, b):
                pltpu.async_copy(s, d, sem)

        boff = b * B
        ob = lax.rem(kb, 2)
        ooff = ob * B

        @plsc.parallel_loop(0, B, 1, unroll=4)
        def ebody(e):
            acc = jnp.zeros((16,), jnp.float32)
            for c in range(PW // 16):
                wa = rows0_v[boff + e, pl.ds(c * 16, 16)]
                wb = rows1_v[boff + e, pl.ds(c * 16, 16)]
                m = plsc.bitcast(wa, jnp.bfloat16) * plsc.bitcast(
                    wb, jnp.bfloat16)
                m0, m1 = plsc.unpack(m, format=plsc.PackFormat.INTERLEAVED)
                acc = acc + m0 + m1
            tot = plsc.cumsum(acc)
            plsc.store_scatter(out_v, [jnp.full((16,), ooff + e, jnp.int32)],
                               tot, mask=last_lane)

        @pl.when(kb >= 1)
        def _wait_prev_out():
            pltpu.make_async_copy(out_v.at[pl.ds((1 - ob) * B, B)],
                                  out_hbm.at[pl.ds(tbase, B)], sem_o).wait()
        pltpu.async_copy(out_v.at[pl.ds(ooff, B)],
                         out_hbm.at[pl.ds(tbase + kb * B, B)], sem_o)
        return 0

    lax.fori_loop(0, NBK, block_body, 0)
    pltpu.make_async_copy(out_v.at[pl.ds(lax.rem(NBK - 1, 2) * B, B)],
                          out_hbm.at[pl.ds(tbase, B)], sem_o).wait()


def kernel(node_feature, edge_label_index, edge_label, W1, b1, W2, b2):
    h_packed = _mlp_packed(node_feature, W1, b1, W2, b2)
    pred = _sc_edge_dot(h_packed, edge_label_index.reshape(-1))
    return pred, edge_label


# speedup vs baseline: 1.1631x; 1.1560x over previous
import functools

import jax
import jax.numpy as jnp
from jax import lax
from jax.experimental import pallas as pl
from jax.experimental.pallas import tpu as pltpu
from jax.experimental.pallas import tpu_sc as plsc

N_NODES = 10000
D = 128
N_EDGES = 320000

NC = 2
NS = 16
NW = NC * NS
PW = D // 2

E_PER_T = N_EDGES // NW
B = 200
NBK = E_PER_T // B
NBUF = 3


def _mlp_body(x_ref, w1_ref, b1_ref, w2_ref, b2_ref, w_ref):
    h1 = jnp.dot(x_ref[...], w1_ref[...], preferred_element_type=jnp.float32)
    h1 = jnp.maximum(h1 + b1_ref[...], 0.0)
    h2 = jnp.dot(h1, w2_ref[...], preferred_element_type=jnp.float32)
    h2 = h2 + b2_ref[...]
    hb = h2.astype(jnp.bfloat16)
    lo = lax.bitcast_convert_type(hb[:, :PW], jnp.uint16).astype(jnp.uint32)
    hi = lax.bitcast_convert_type(hb[:, PW:], jnp.uint16).astype(jnp.uint32)
    w_ref[...] = lax.bitcast_convert_type(lo | (hi << 16), jnp.float32)


def _mlp_packed(node_feature, W1, b1, W2, b2):
    return pl.pallas_call(
        _mlp_body,
        out_shape=jax.ShapeDtypeStruct((N_NODES, PW), jnp.float32),
    )(node_feature, W1, b1.reshape(1, D), W2, b2.reshape(1, D))


_sc_mesh = plsc.VectorSubcoreMesh(core_axis_name="c", subcore_axis_name="s")


@functools.partial(
    pl.kernel,
    out_type=jax.ShapeDtypeStruct((N_EDGES,), jnp.float32),
    mesh=_sc_mesh,
    scratch_types=[
        pltpu.VMEM((NBUF * B, PW), jnp.float32),
        pltpu.VMEM((NBUF * B, PW), jnp.float32),
        pltpu.VMEM((NBUF * B,), jnp.int32),
        pltpu.VMEM((NBUF * B,), jnp.int32),
        pltpu.VMEM((2 * B,), jnp.float32),
        pltpu.VMEM_SHARED((N_NODES, PW), jnp.float32),
        pltpu.SemaphoreType.DMA,
        pltpu.SemaphoreType.DMA,
        pltpu.SemaphoreType.DMA,
        pltpu.SemaphoreType.DMA,
        pltpu.SemaphoreType.DMA,
    ],
    compiler_params=pltpu.CompilerParams(needs_layout_passes=False,
                                         use_tc_tiling_on_sc=False),
)
def _sc_edge_dot(h_packed, idx_flat, out_hbm,
                 rows0_v, rows1_v, idx0_v, idx1_v, out_v, table_sh,
                 sem_i0, sem_i1, sem_r0, sem_r1, sem_o):
    t = lax.axis_index("s") * NC + lax.axis_index("c")
    s_id = lax.axis_index("s")
    iota16 = lax.iota(jnp.int32, 16)
    last_lane = iota16 == 15
    tbase = t * E_PER_T

    def idx_copies(kb, buf):
        src0 = idx_flat.at[pl.ds(tbase + kb * B, B)]
        src1 = idx_flat.at[pl.ds(N_EDGES + tbase + kb * B, B)]
        d0 = idx0_v.at[pl.ds(buf * B, B)]
        d1 = idx1_v.at[pl.ds(buf * B, B)]
        return ((src0, d0, sem_i0), (src1, d1, sem_i1))

    def row_copies(buf):
        i0 = idx0_v.at[pl.ds(buf * B, B)]
        i1 = idx1_v.at[pl.ds(buf * B, B)]
        return ((h_packed.at[i0], rows0_v.at[pl.ds(buf * B, B)], sem_r0),
                (table_sh.at[i1], rows1_v.at[pl.ds(buf * B, B)], sem_r1))

    for k in range(NBUF):
        for s, d, sem in idx_copies(k, k):
            pltpu.async_copy(s, d, sem)

    for j in range(5):
        rslice = pl.ds(s_id * 625 + j * 125, 125)
        pltpu.sync_copy(h_packed.at[rslice], rows0_v.at[pl.ds(0, 125)])
        pltpu.sync_copy(rows0_v.at[pl.ds(0, 125)], table_sh.at[rslice])
    plsc.subcore_barrier()

    for k in range(2):
        for s, d, sem in idx_copies(k, k):
            pltpu.make_async_copy(s, d, sem).wait()
        for s, d, sem in row_copies(k):
            pltpu.async_copy(s, d, sem)

    def block_body(kb, _):
        b = lax.rem(kb, NBUF)
        b2 = lax.rem(kb + 2, NBUF)

        @pl.when(kb + 2 < NBK)
        def _start_ahead_rows():
            for s, d, sem in idx_copies(kb + 2, b2):
                pltpu.make_async_copy(s, d, sem).wait()
            for s, d, sem in row_copies(b2):
                pltpu.async_copy(s, d, sem)

        for s, d, sem in row_copies(b):
            pltpu.make_async_copy(s, d, sem).wait()

        @pl.when(kb + NBUF < NBK)
        def _prefetch_idx():
            for s, d, sem in idx_copies(kb + NBUF, b):
                pltpu.async_copy(s, d, sem)

        boff = b * B
        ob = lax.rem(kb, 2)
        ooff = ob * B

        @plsc.parallel_loop(0, B, 1, unroll=4)
        def ebody(e):
            acc = jnp.zeros((16,), jnp.float32)
            for c in range(PW // 16):
                wa = rows0_v[boff + e, pl.ds(c * 16, 16)]
                wb = rows1_v[boff + e, pl.ds(c * 16, 16)]
                m = plsc.bitcast(wa, jnp.bfloat16) * plsc.bitcast(
                    wb, jnp.bfloat16)
                m0, m1 = plsc.unpack(m, format=plsc.PackFormat.INTERLEAVED)
                acc = acc + m0 + m1
            tot = plsc.cumsum(acc)
            plsc.store_scatter(out_v, [jnp.full((16,), ooff + e, jnp.int32)],
                               tot, mask=last_lane)

        @pl.when(kb >= 1)
        def _wait_prev_out():
            pltpu.make_async_copy(out_v.at[pl.ds((1 - ob) * B, B)],
                                  out_hbm.at[pl.ds(tbase, B)], sem_o).wait()
        pltpu.async_copy(out_v.at[pl.ds(ooff, B)],
                         out_hbm.at[pl.ds(tbase + kb * B, B)], sem_o)
        return 0

    lax.fori_loop(0, NBK, block_body, 0)
    pltpu.make_async_copy(out_v.at[pl.ds(lax.rem(NBK - 1, 2) * B, B)],
                          out_hbm.at[pl.ds(tbase, B)], sem_o).wait()


def kernel(node_feature, edge_label_index, edge_label, W1, b1, W2, b2):
    h_packed = _mlp_packed(node_feature, W1, b1, W2, b2)
    pred = _sc_edge_dot(h_packed, edge_label_index.reshape(-1))
    return pred, edge_label


# speedup vs baseline: 1.1799x; 1.0144x over previous
import functools

import jax
import jax.numpy as jnp
from jax import lax
from jax.experimental import pallas as pl
from jax.experimental.pallas import tpu as pltpu
from jax.experimental.pallas import tpu_sc as plsc

N_NODES = 10000
D = 128
N_EDGES = 320000

NC = 2
NS = 16
NW = NC * NS
PW = D // 2

E_PER_T = N_EDGES // NW
B = 200
NBK = E_PER_T // B
NBUF = 3


def _mlp_body(x_ref, w1_ref, b1_ref, w2_ref, b2_ref, w_ref):
    h1 = jnp.dot(x_ref[...], w1_ref[...], preferred_element_type=jnp.float32)
    h1 = jnp.maximum(h1 + b1_ref[...], 0.0)
    h2 = jnp.dot(h1, w2_ref[...], preferred_element_type=jnp.float32)
    h2 = h2 + b2_ref[...]
    hb = h2.astype(jnp.bfloat16)
    lo = lax.bitcast_convert_type(hb[:, :PW], jnp.uint16).astype(jnp.uint32)
    hi = lax.bitcast_convert_type(hb[:, PW:], jnp.uint16).astype(jnp.uint32)
    w_ref[...] = lax.bitcast_convert_type(lo | (hi << 16), jnp.float32)


def _mlp_packed(node_feature, W1, b1, W2, b2):
    return pl.pallas_call(
        _mlp_body,
        out_shape=jax.ShapeDtypeStruct((N_NODES, PW), jnp.float32),
    )(node_feature, W1, b1.reshape(1, D), W2, b2.reshape(1, D))


_sc_mesh = plsc.VectorSubcoreMesh(core_axis_name="c", subcore_axis_name="s")


@functools.partial(
    pl.kernel,
    out_type=jax.ShapeDtypeStruct((N_EDGES,), jnp.float32),
    mesh=_sc_mesh,
    scratch_types=[
        pltpu.VMEM((NBUF * B, PW), jnp.float32),
        pltpu.VMEM((NBUF * B, PW), jnp.float32),
        pltpu.VMEM((NBUF * B,), jnp.int32),
        pltpu.VMEM((NBUF * B,), jnp.int32),
        pltpu.VMEM((2 * B,), jnp.float32),
        pltpu.VMEM_SHARED((N_NODES, PW), jnp.float32),
        pltpu.SemaphoreType.DMA,
        pltpu.SemaphoreType.DMA,
        pltpu.SemaphoreType.DMA,
        pltpu.SemaphoreType.DMA,
        pltpu.SemaphoreType.DMA,
    ],
    compiler_params=pltpu.CompilerParams(needs_layout_passes=False,
                                         use_tc_tiling_on_sc=False),
)
def _sc_edge_dot(h_packed, idx_flat, out_hbm,
                 rows0_v, rows1_v, idx0_v, idx1_v, out_v, table_sh,
                 sem_i0, sem_i1, sem_r0, sem_r1, sem_o):
    t = lax.axis_index("s") * NC + lax.axis_index("c")
    s_id = lax.axis_index("s")
    iota16 = lax.iota(jnp.int32, 16)
    last_lane = iota16 == 15
    tbase = t * E_PER_T

    def idx_copies(kb, buf):
        src0 = idx_flat.at[pl.ds(tbase + kb * B, B)]
        src1 = idx_flat.at[pl.ds(N_EDGES + tbase + kb * B, B)]
        d0 = idx0_v.at[pl.ds(buf * B, B)]
        d1 = idx1_v.at[pl.ds(buf * B, B)]
        return ((src0, d0, sem_i0), (src1, d1, sem_i1))

    def row_copies(buf):
        i0 = idx0_v.at[pl.ds(buf * B, B)]
        i1 = idx1_v.at[pl.ds(buf * B, B)]
        return ((h_packed.at[i0], rows0_v.at[pl.ds(buf * B, B)], sem_r0),
                (table_sh.at[i1], rows1_v.at[pl.ds(buf * B, B)], sem_r1))

    for k in range(NBUF):
        for s, d, sem in idx_copies(k, k):
            pltpu.async_copy(s, d, sem)

    bounce = (rows0_v.at[pl.ds(2 * B, 125)], rows1_v.at[pl.ds(2 * B, 125)])
    bsem = (sem_r0, sem_r1)

    def _piece(j):
        return pl.ds(s_id * 625 + j * 125, 125)

    pltpu.async_copy(h_packed.at[_piece(0)], bounce[0], bsem[0])
    for j in range(5):
        w = j % 2
        pltpu.make_async_copy(h_packed.at[_piece(j)], bounce[w],
                              bsem[w]).wait()
        if j + 1 < 5:
            pltpu.async_copy(h_packed.at[_piece(j + 1)], bounce[1 - w],
                             bsem[1 - w])
        pltpu.sync_copy(bounce[w], table_sh.at[_piece(j)])
    plsc.subcore_barrier()

    for k in range(2):
        for s, d, sem in idx_copies(k, k):
            pltpu.make_async_copy(s, d, sem).wait()
        for s, d, sem in row_copies(k):
            pltpu.async_copy(s, d, sem)

    def block_body(kb, _):
        b = lax.rem(kb, NBUF)
        b2 = lax.rem(kb + 2, NBUF)

        @pl.when(kb + 2 < NBK)
        def _start_ahead_rows():
            for s, d, sem in idx_copies(kb + 2, b2):
                pltpu.make_async_copy(s, d, sem).wait()
            for s, d, sem in row_copies(b2):
                pltpu.async_copy(s, d, sem)

        for s, d, sem in row_copies(b):
            pltpu.make_async_copy(s, d, sem).wait()

        @pl.when(kb + NBUF < NBK)
        def _prefetch_idx():
            for s, d, sem in idx_copies(kb + NBUF, b):
                pltpu.async_copy(s, d, sem)

        boff = b * B
        ob = lax.rem(kb, 2)
        ooff = ob * B

        @plsc.parallel_loop(0, B, 1, unroll=4)
        def ebody(e):
            acc = jnp.zeros((16,), jnp.float32)
            for c in range(PW // 16):
                wa = rows0_v[boff + e, pl.ds(c * 16, 16)]
                wb = rows1_v[boff + e, pl.ds(c * 16, 16)]
                m = plsc.bitcast(wa, jnp.bfloat16) * plsc.bitcast(
                    wb, jnp.bfloat16)
                m0, m1 = plsc.unpack(m, format=plsc.PackFormat.INTERLEAVED)
                acc = acc + m0 + m1
            tot = plsc.cumsum(acc)
            plsc.store_scatter(out_v, [jnp.full((16,), ooff + e, jnp.int32)],
                               tot, mask=last_lane)

        @pl.when(kb >= 1)
        def _wait_prev_out():
            pltpu.make_async_copy(out_v.at[pl.ds((1 - ob) * B, B)],
                                  out_hbm.at[pl.ds(tbase, B)], sem_o).wait()
        pltpu.async_copy(out_v.at[pl.ds(ooff, B)],
                         out_hbm.at[pl.ds(tbase + kb * B, B)], sem_o)
        return 0

    lax.fori_loop(0, NBK, block_body, 0)
    pltpu.make_async_copy(out_v.at[pl.ds(lax.rem(NBK - 1, 2) * B, B)],
                          out_hbm.at[pl.ds(tbase, B)], sem_o).wait()


def kernel(node_feature, edge_label_index, edge_label, W1, b1, W2, b2):
    h_packed = _mlp_packed(node_feature, W1, b1, W2, b2)
    pred = _sc_edge_dot(h_packed, edge_label_index.reshape(-1))
    return pred, edge_label
